# baseline (speedup 1.0000x reference)
import jax
import jax.numpy as jnp
from jax.experimental import pallas as pl


def _body(col_ref, row_ref, out_ref):
    _, C, HW = out_ref.shape
    D = C // 2
    H = W = 32
    colT = col_ref[:W, :].T
    rowT = row_ref[:H, :].T
    x_part = jnp.broadcast_to(colT[:, None, :], (D, H, W)).reshape(D, HW)
    y_part = jnp.broadcast_to(rowT[:, :, None], (D, H, W)).reshape(D, HW)
    pos = jnp.concatenate([x_part, y_part], axis=0)
    out_ref[...] = pos[None]


def kernel(input_, row_embed, col_embed):
    B, _, H, W = input_.shape
    D = row_embed.shape[1]
    C = 2 * D
    out = pl.pallas_call(
        _body,
        grid=(B,),
        in_specs=[
            pl.BlockSpec((50, D), lambda b: (0, 0)),
            pl.BlockSpec((50, D), lambda b: (0, 0)),
        ],
        out_specs=pl.BlockSpec((1, C, H * W), lambda b: (b, 0, 0)),
        out_shape=jax.ShapeDtypeStruct((B, C, H * W), jnp.float32),
    )(col_embed, row_embed)
    return out.reshape(B, C, H, W)


# speedup vs baseline: 1.5109x; 1.5109x over previous
import jax
import jax.numpy as jnp
from jax.experimental import pallas as pl
from jax.experimental.pallas import tpu as pltpu


def _body(col_ref, row_ref, out_ref, pos_vmem, sem):
    B, C, HW = out_ref.shape
    D = C // 2
    H = W = 32
    colT = col_ref[:W, :].T
    rowT = row_ref[:H, :].T
    x_part = jnp.broadcast_to(colT[:, None, :], (D, H, W)).reshape(D, HW)
    y_part = jnp.broadcast_to(rowT[:, :, None], (D, H, W)).reshape(D, HW)
    pos_vmem[...] = jnp.concatenate([x_part, y_part], axis=0)
    copies = [
        pltpu.make_async_copy(pos_vmem, out_ref.at[b], sem) for b in range(B)
    ]
    for cp in copies:
        cp.start()
    for cp in copies:
        cp.wait()


def kernel(input_, row_embed, col_embed):
    B, _, H, W = input_.shape
    D = row_embed.shape[1]
    C = 2 * D
    out = pl.pallas_call(
        _body,
        in_specs=[
            pl.BlockSpec(memory_space=pltpu.VMEM),
            pl.BlockSpec(memory_space=pltpu.VMEM),
        ],
        out_specs=pl.BlockSpec(memory_space=pl.ANY),
        out_shape=jax.ShapeDtypeStruct((B, C, H * W), jnp.float32),
        scratch_shapes=[
            pltpu.VMEM((C, H * W), jnp.float32),
            pltpu.SemaphoreType.DMA,
        ],
    )(col_embed, row_embed)
    return out.reshape(B, C, H, W)
